# Initial kernel scaffold; baseline (speedup 1.0000x reference)
import jax
import jax.numpy as jnp
from jax.experimental import pallas as pl


def kernel(x, edge_index_sc, edge_index_fc, batch, Wl1, bl1, Wr1, g1, beta1, Wl2, bl2, Wr2, g2, beta2, Wl3, bl3, Wr3, Wout, bout):
    raise NotImplementedError("write your pallas kernel here")


# speedup vs baseline: 6.6485x; 6.6485x over previous
import jax
import jax.numpy as jnp
from jax import lax
from jax.experimental import pallas as pl
from jax.experimental.pallas import tpu as pltpu
from jax.experimental.pallas import tpu_sc as plsc

N = 10000
E = 320000
H = 128
G = 8
NP = 10240
NTILE = 16
RPT = NP // NTILE
EPT = E // NTILE
K = 160
F32 = jnp.float32
HIGH = lax.Precision.HIGHEST

_mesh = plsc.VectorSubcoreMesh(core_axis_name="c", subcore_axis_name="s")


ZR = 80


def _fill_1d(ref, length, val):
    @pl.loop(0, length, step=16)
    def _(i):
        ref[pl.ds(i, 16)] = jnp.full((16,), val, F32)


def _zero_rows(ref, nrows):
    @pl.loop(0, nrows)
    def _(i):
        for j in range(H // 16):
            ref[i, pl.ds(16 * j, 16)] = jnp.zeros((16,), F32)


def _make_agg(with_deg, offs_mult):
    outs = [jax.ShapeDtypeStruct((2, NP, H), F32)]
    if with_deg:
        outs.append(jax.ShapeDtypeStruct((2 * NP,), F32))
    scratch = [
        pltpu.VMEM((K,), jnp.int32),
        pltpu.VMEM((K,), jnp.int32),
        pltpu.VMEM((K, H), F32),
        pltpu.VMEM((K,), F32),
        pltpu.VMEM((RPT,), F32),
        pltpu.VMEM_SHARED((NP, H), F32),
        pltpu.VMEM_SHARED((NP,), F32),
        pltpu.SemaphoreType.DMA,
    ]

    def body(tab, edges, *rest):
        if with_deg:
            sums, deg, srcv, dstv, rows, onesv, zv, acc_sh, deg_sh, sem = rest
        else:
            sums, srcv, dstv, rows, onesv, zv, acc_sh, deg_sh, sem = rest
        c = lax.axis_index("c")
        s = lax.axis_index("s")
        _zero_rows(rows, ZR)
        for r in range(RPT // ZR):
            pltpu.sync_copy(rows.at[pl.ds(0, ZR)],
                            acc_sh.at[pl.ds(s * RPT + ZR * r, ZR)])
        if with_deg:
            _fill_1d(zv, RPT, 0.0)
            pltpu.sync_copy(zv, deg_sh.at[pl.ds(s * RPT, RPT)])
            _fill_1d(onesv, K, 1.0)
        plsc.subcore_barrier()
        src_base = (2 * c) * E + s * EPT
        dst_base = (2 * c + 1) * E + s * EPT

        @pl.loop(0, EPT, step=K)
        def _(off):
            pltpu.sync_copy(edges.at[pl.ds(src_base + off, K)], srcv)
            pltpu.sync_copy(edges.at[pl.ds(dst_base + off, K)], dstv)
            if offs_mult:
                shift = c * offs_mult

                @pl.loop(0, K, step=16)
                def _(j):
                    srcv[pl.ds(j, 16)] = srcv[pl.ds(j, 16)] + shift

            pltpu.async_copy(tab.at[srcv], rows, sem).wait()
            pltpu.sync_copy(rows, acc_sh.at[dstv], add=True)
            if with_deg:
                pltpu.sync_copy(onesv, deg_sh.at[dstv], add=True)

        plsc.subcore_barrier()
        pltpu.sync_copy(acc_sh.at[pl.ds(s * RPT, RPT)],
                        sums.at[c, pl.ds(s * RPT, RPT)])
        if with_deg:
            pltpu.sync_copy(deg_sh.at[pl.ds(s * RPT, RPT)],
                            deg.at[pl.ds(c * NP + s * RPT, RPT)])

    return pl.kernel(body, out_type=tuple(outs) if with_deg else outs[0],
                     mesh=_mesh, scratch_types=scratch)


def _make_sagg():
    scratch = [
        pltpu.VMEM((K,), jnp.int32),
        pltpu.VMEM((K,), jnp.int32),
        pltpu.VMEM((K,), F32),
        pltpu.VMEM((RPT,), F32),
        pltpu.VMEM_SHARED((NP,), F32),
        pltpu.SemaphoreType.DMA,
    ]

    def body(ztab, edges, t_out, srcv, dstv, valv, zv, t_sh, sem):
        c = lax.axis_index("c")
        s = lax.axis_index("s")
        _fill_1d(zv, RPT, 0.0)
        pltpu.sync_copy(zv, t_sh.at[pl.ds(s * RPT, RPT)])
        plsc.subcore_barrier()
        src_base = (2 * c) * E + s * EPT
        dst_base = (2 * c + 1) * E + s * EPT

        @pl.loop(0, EPT, step=K)
        def _(off):
            pltpu.sync_copy(edges.at[pl.ds(src_base + off, K)], srcv)
            pltpu.sync_copy(edges.at[pl.ds(dst_base + off, K)], dstv)
            shift = c * NP

            @pl.loop(0, K, step=16)
            def _(j):
                srcv[pl.ds(j, 16)] = srcv[pl.ds(j, 16)] + shift

            pltpu.async_copy(ztab.at[srcv], valv, sem).wait()
            pltpu.sync_copy(valv, t_sh.at[dstv], add=True)

        plsc.subcore_barrier()
        pltpu.sync_copy(t_sh.at[pl.ds(s * RPT, RPT)],
                        t_out.at[pl.ds(c * NP + s * RPT, RPT)])

    return pl.kernel(body, out_type=jax.ShapeDtypeStruct((2 * NP,), F32),
                     mesh=_mesh, scratch_types=scratch)


_agg1 = _make_agg(with_deg=True, offs_mult=0)
_agg2 = _make_agg(with_deg=False, offs_mult=NP)
_sagg = _make_sagg()


RB = 1024
NRB = NP // RB


def _rowmask(j):
    base = j * RB
    ri = lax.broadcasted_iota(jnp.int32, (RB, 1), 0) + base
    return jnp.where(ri < N, 1.0, 0.0)


def _lin_body(sums, deg, h, Wl, bl, Wr, g, beta, y_out, stats_out):
    j = pl.program_id(1)
    inv = 1.0 / jnp.maximum(deg[0], 1.0)
    y = lax.dot_general(sums[0], Wl[...], (((1,), (1,)), ((), ())),
                        precision=HIGH) * inv
    y = y + bl[...] + lax.dot_general(h[0], Wr[...], (((1,), (1,)), ((), ())),
                                      precision=HIGH)
    y_out[0] = y
    ym = y * _rowmask(j)

    @pl.when(j == 0)
    def _():
        stats_out[...] = jnp.zeros_like(stats_out)

    stats_out[0, 0, :] += jnp.sum(ym, axis=0)
    stats_out[0, 1, :] += jnp.sum(ym * ym, axis=0)


def _bn_body(y, stats, g, beta, out):
    j = pl.program_id(1)
    m = stats[0, 0, :] * (1.0 / N)
    v = stats[0, 1, :] * (1.0 / N) - m * m
    h = jnp.maximum((y[0] - m) * lax.rsqrt(v + 1e-5) * g[...] + beta[...], 0.0)
    out[0] = h * _rowmask(j)


def _bnproj_body(y, stats, g, beta, Wl3, Wr3, Wout, z_out, w_out):
    j = pl.program_id(1)
    m = stats[0, 0, :] * (1.0 / N)
    v = stats[0, 1, :] * (1.0 / N) - m * m
    h = jnp.maximum((y[0] - m) * lax.rsqrt(v + 1e-5) * g[...] + beta[...], 0.0)
    h = h * _rowmask(j)
    u3 = lax.dot_general(Wl3[...], Wout[...], (((0,), (1,)), ((), ())),
                         precision=HIGH)
    v3 = lax.dot_general(Wr3[...], Wout[...], (((0,), (1,)), ((), ())),
                         precision=HIGH)
    z_out[0] = lax.dot_general(h, u3, (((1,), (0,)), ((), ())), precision=HIGH)
    w_out[0] = lax.dot_general(h, v3, (((1,), (0,)), ((), ())), precision=HIGH)


def _wspec():
    return pl.BlockSpec((H, H), lambda t, j: (0, 0))


def _vspec():
    return pl.BlockSpec((H,), lambda t, j: (0,))


def _blkspec(w):
    return pl.BlockSpec((1, RB, w), lambda t, j: (t, j, 0))


def _statspec():
    return pl.BlockSpec((1, 2, H), lambda t, j: (t, 0, 0))


def _lin_call(h_spec):
    return pl.pallas_call(
        _lin_body,
        grid=(2, NRB),
        in_specs=[_blkspec(H), _blkspec(1), h_spec,
                  _wspec(), _vspec(), _wspec(), _vspec(), _vspec()],
        out_specs=[_blkspec(H), _statspec()],
        out_shape=[jax.ShapeDtypeStruct((2, NP, H), F32),
                   jax.ShapeDtypeStruct((2, 2, H), F32)],
    )


def _final_body(t, deg, w, batch, bl3, Wout, bout, out):
    c3 = jnp.sum(bl3[...] * Wout[0, :])
    node = 0.5 * (t[0] / jnp.maximum(deg[0], 1.0)
                  + t[1] / jnp.maximum(deg[1], 1.0)
                  + w[0] + w[1]) + c3
    b = batch[...]
    gi = lax.broadcasted_iota(jnp.int32, (G, 1), 0)
    acc = jnp.zeros((G, 1), F32)
    for g in range(G):
        mask = b == g
        sg = jnp.sum(jnp.where(mask, node, 0.0))
        cg = jnp.sum(jnp.where(mask, 1.0, 0.0))
        acc = acc + jnp.where(gi == g, sg / jnp.maximum(cg, 1.0), 0.0)
    out[...] = acc + bout[...]


def kernel(x, edge_index_sc, edge_index_fc, batch,
           Wl1, bl1, Wr1, g1, beta1,
           Wl2, bl2, Wr2, g2, beta2,
           Wl3, bl3, Wr3, Wout, bout):
    edges = jnp.stack([edge_index_sc, edge_index_fc]).reshape(4 * E)
    x2 = jnp.pad(x, ((0, NP - N), (0, 0)))[None]

    sums1, deg = _agg1(x2.reshape(NP, H), edges)
    deg3 = deg.reshape(2, NP, 1)

    y1, st1 = _lin_call(pl.BlockSpec((1, RB, H), lambda t, j: (0, j, 0)))(
        sums1, deg3, x2, Wl1, bl1, Wr1, g1, beta1)
    bn1 = pl.pallas_call(
        _bn_body,
        grid=(2, NRB),
        in_specs=[_blkspec(H), _statspec(), _vspec(), _vspec()],
        out_specs=_blkspec(H),
        out_shape=jax.ShapeDtypeStruct((2, NP, H), F32),
    )
    h1 = bn1(y1, st1, g1, beta1)

    sums2 = _agg2(h1.reshape(2 * NP, H), edges)

    y2, st2 = _lin_call(_blkspec(H))(
        sums2, deg3, h1, Wl2, bl2, Wr2, g2, beta2)
    bnproj = pl.pallas_call(
        _bnproj_body,
        grid=(2, NRB),
        in_specs=[_blkspec(H), _statspec(), _vspec(), _vspec(),
                  _wspec(), _wspec(),
                  pl.BlockSpec((1, H), lambda t, j: (0, 0))],
        out_specs=[_blkspec(1), _blkspec(1)],
        out_shape=[jax.ShapeDtypeStruct((2, NP, 1), F32),
                   jax.ShapeDtypeStruct((2, NP, 1), F32)],
    )
    z, w = bnproj(y2, st2, g2, beta2, Wl3, Wr3, Wout)

    t = _sagg(z.reshape(2 * NP), edges)

    batch_p = jnp.pad(batch, (0, NP - N), constant_values=G)
    final = pl.pallas_call(
        _final_body,
        out_shape=jax.ShapeDtypeStruct((G, 1), F32),
    )
    return final(t.reshape(2, NP // H, H), deg.reshape(2, NP // H, H),
                 w.reshape(2, NP // H, H), batch_p.reshape(NP // H, H),
                 bl3, Wout, bout.reshape(1, 1))


# speedup vs baseline: 11.5209x; 1.7329x over previous
import jax
import jax.numpy as jnp
from jax import lax
from jax.experimental import pallas as pl
from jax.experimental.pallas import tpu as pltpu
from jax.experimental.pallas import tpu_sc as plsc

N = 10000
E = 320000
H = 128
G = 8
NP = 10240
NTILE = 16
RPT = NP // NTILE
EPT = E // NTILE
K = 160
F32 = jnp.float32
HIGH = lax.Precision.HIGHEST

_mesh = plsc.VectorSubcoreMesh(core_axis_name="c", subcore_axis_name="s")


ZR = 80


def _fill_1d(ref, length, val):
    @pl.loop(0, length, step=16)
    def _(i):
        ref[pl.ds(i, 16)] = jnp.full((16,), val, F32)


def _zero_rows(ref, nrows):
    @pl.loop(0, nrows)
    def _(i):
        for j in range(H // 16):
            ref[i, pl.ds(16 * j, 16)] = jnp.zeros((16,), F32)


def _pipeline_chunks(nchunks, load_idx, gather, scatter, wait_gather,
                     wait_scatter):
    def turn(q, b, first):
        if not first:
            wait_scatter(b)
        load_idx(q, b)
        gather(q, b)
        if not (first and b == 0):
            wait_gather(1 - b)
            scatter(q - 1, 1 - b)

    turn(0, 0, True)
    turn(1, 1, True)
    npeel = 2 + ((nchunks - 2) % 2)
    for q in range(2, npeel):
        turn(q, q % 2, False)

    @pl.loop(0, nchunks - npeel, step=2)
    def _(i):
        for k in range(2):
            turn(npeel + i + k, (npeel + k) % 2, False)

    b_last = (nchunks - 1) % 2
    wait_gather(b_last)
    scatter(nchunks - 1, b_last)
    wait_scatter(1 - b_last)
    wait_scatter(b_last)


def _make_agg(with_deg, offs_mult):
    outs = [jax.ShapeDtypeStruct((2, NP, H), F32)]
    if with_deg:
        outs.append(jax.ShapeDtypeStruct((2 * NP,), F32))
    scratch = [
        pltpu.VMEM((K,), jnp.int32),
        pltpu.VMEM((K,), jnp.int32),
        pltpu.VMEM((K,), jnp.int32),
        pltpu.VMEM((K,), jnp.int32),
        pltpu.VMEM((K, H), F32),
        pltpu.VMEM((K, H), F32),
        pltpu.VMEM((K,), F32),
        pltpu.VMEM((RPT,), F32),
        pltpu.VMEM_SHARED((NP, H), F32),
        pltpu.VMEM_SHARED((NP,), F32),
        pltpu.SemaphoreType.DMA,
        pltpu.SemaphoreType.DMA,
        pltpu.SemaphoreType.DMA,
        pltpu.SemaphoreType.DMA,
    ]

    def body(tab, edges, *rest):
        if with_deg:
            (sums, deg, srcv0, srcv1, dstv0, dstv1, rows0, rows1, onesv, zv,
             acc_sh, deg_sh, sg0, sg1, ss0, ss1) = rest
        else:
            (sums, srcv0, srcv1, dstv0, dstv1, rows0, rows1, onesv, zv,
             acc_sh, deg_sh, sg0, sg1, ss0, ss1) = rest
        srcv = (srcv0, srcv1)
        dstv = (dstv0, dstv1)
        rows = (rows0, rows1)
        sg = (sg0, sg1)
        ss = (ss0, ss1)
        c = lax.axis_index("c")
        s = lax.axis_index("s")
        _zero_rows(rows0, ZR)
        for r in range(RPT // ZR):
            pltpu.sync_copy(rows0.at[pl.ds(0, ZR)],
                            acc_sh.at[pl.ds(s * RPT + ZR * r, ZR)])
        if with_deg:
            _fill_1d(zv, RPT, 0.0)
            pltpu.sync_copy(zv, deg_sh.at[pl.ds(s * RPT, RPT)])
            _fill_1d(onesv, K, 1.0)
        plsc.subcore_barrier()
        src_base = (2 * c) * E + s * EPT
        dst_base = (2 * c + 1) * E + s * EPT
        shift = c * offs_mult

        def load_idx(q, b):
            off = q * K
            pltpu.sync_copy(edges.at[pl.ds(src_base + off, K)], srcv[b])
            pltpu.sync_copy(edges.at[pl.ds(dst_base + off, K)], dstv[b])
            if offs_mult:
                @pl.loop(0, K, step=16)
                def _(j):
                    srcv[b][pl.ds(j, 16)] = srcv[b][pl.ds(j, 16)] + shift

        def gather(q, b):
            pltpu.async_copy(tab.at[srcv[b]], rows[b], sg[b])

        def wait_gather(b):
            pltpu.make_async_copy(tab.at[srcv[b]], rows[b], sg[b]).wait()

        def scatter(q, b):
            pltpu.async_copy(rows[b], acc_sh.at[dstv[b]], ss[b], add=True)
            if with_deg:
                pltpu.async_copy(onesv, deg_sh.at[dstv[b]], ss[b], add=True)

        def wait_scatter(b):
            pltpu.make_async_copy(rows[b], acc_sh.at[dstv[b]], ss[b]).wait()
            if with_deg:
                pltpu.make_async_copy(onesv, deg_sh.at[dstv[b]],
                                      ss[b]).wait()

        _pipeline_chunks(EPT // K, load_idx, gather, scatter, wait_gather,
                         wait_scatter)

        plsc.subcore_barrier()
        pltpu.sync_copy(acc_sh.at[pl.ds(s * RPT, RPT)],
                        sums.at[c, pl.ds(s * RPT, RPT)])
        if with_deg:
            pltpu.sync_copy(deg_sh.at[pl.ds(s * RPT, RPT)],
                            deg.at[pl.ds(c * NP + s * RPT, RPT)])

    return pl.kernel(body, out_type=tuple(outs) if with_deg else outs[0],
                     mesh=_mesh, scratch_types=scratch)


KS = 2000


def _make_sagg():
    scratch = [
        pltpu.VMEM((KS,), jnp.int32),
        pltpu.VMEM((KS,), jnp.int32),
        pltpu.VMEM((KS,), jnp.int32),
        pltpu.VMEM((KS,), jnp.int32),
        pltpu.VMEM((KS,), F32),
        pltpu.VMEM((KS,), F32),
        pltpu.VMEM((RPT,), F32),
        pltpu.VMEM_SHARED((NP,), F32),
        pltpu.SemaphoreType.DMA,
        pltpu.SemaphoreType.DMA,
        pltpu.SemaphoreType.DMA,
        pltpu.SemaphoreType.DMA,
    ]

    def body(ztab, edges, t_out, srcv0, srcv1, dstv0, dstv1, val0, val1, zv,
             t_sh, sg0, sg1, ss0, ss1):
        srcv = (srcv0, srcv1)
        dstv = (dstv0, dstv1)
        vals = (val0, val1)
        sg = (sg0, sg1)
        ss = (ss0, ss1)
        c = lax.axis_index("c")
        s = lax.axis_index("s")
        _fill_1d(zv, RPT, 0.0)
        pltpu.sync_copy(zv, t_sh.at[pl.ds(s * RPT, RPT)])
        plsc.subcore_barrier()
        src_base = (2 * c) * E + s * EPT
        dst_base = (2 * c + 1) * E + s * EPT
        shift = c * NP

        def load_idx(q, b):
            off = q * KS
            pltpu.sync_copy(edges.at[pl.ds(src_base + off, KS)], srcv[b])
            pltpu.sync_copy(edges.at[pl.ds(dst_base + off, KS)], dstv[b])

            @pl.loop(0, KS, step=16)
            def _(j):
                srcv[b][pl.ds(j, 16)] = srcv[b][pl.ds(j, 16)] + shift

        def gather(q, b):
            pltpu.async_copy(ztab.at[srcv[b]], vals[b], sg[b])

        def wait_gather(b):
            pltpu.make_async_copy(ztab.at[srcv[b]], vals[b], sg[b]).wait()

        def scatter(q, b):
            pltpu.async_copy(vals[b], t_sh.at[dstv[b]], ss[b], add=True)

        def wait_scatter(b):
            pltpu.make_async_copy(vals[b], t_sh.at[dstv[b]], ss[b]).wait()

        _pipeline_chunks(EPT // KS, load_idx, gather, scatter, wait_gather,
                         wait_scatter)

        plsc.subcore_barrier()
        pltpu.sync_copy(t_sh.at[pl.ds(s * RPT, RPT)],
                        t_out.at[pl.ds(c * NP + s * RPT, RPT)])

    return pl.kernel(body, out_type=jax.ShapeDtypeStruct((2 * NP,), F32),
                     mesh=_mesh, scratch_types=scratch)


_agg1 = _make_agg(with_deg=True, offs_mult=0)
_agg2 = _make_agg(with_deg=False, offs_mult=NP)
_sagg = _make_sagg()


RB = 1024
NRB = NP // RB


def _rowmask(j):
    base = j * RB
    ri = lax.broadcasted_iota(jnp.int32, (RB, 1), 0) + base
    return jnp.where(ri < N, 1.0, 0.0)


def _lin_body(sums, deg, h, Wl, bl, Wr, g, beta, y_out, stats_out):
    j = pl.program_id(1)
    inv = 1.0 / jnp.maximum(deg[0], 1.0)
    y = lax.dot_general(sums[0], Wl[...], (((1,), (1,)), ((), ())),
                        precision=HIGH) * inv
    y = y + bl[...] + lax.dot_general(h[0], Wr[...], (((1,), (1,)), ((), ())),
                                      precision=HIGH)
    y_out[0] = y
    ym = y * _rowmask(j)

    @pl.when(j == 0)
    def _():
        stats_out[...] = jnp.zeros_like(stats_out)

    stats_out[0, 0, :] += jnp.sum(ym, axis=0)
    stats_out[0, 1, :] += jnp.sum(ym * ym, axis=0)


def _bn_body(y, stats, g, beta, out):
    j = pl.program_id(1)
    m = stats[0, 0, :] * (1.0 / N)
    v = stats[0, 1, :] * (1.0 / N) - m * m
    h = jnp.maximum((y[0] - m) * lax.rsqrt(v + 1e-5) * g[...] + beta[...], 0.0)
    out[0] = h * _rowmask(j)


def _bnproj_body(y, stats, g, beta, Wl3, Wr3, Wout, z_out, w_out):
    j = pl.program_id(1)
    m = stats[0, 0, :] * (1.0 / N)
    v = stats[0, 1, :] * (1.0 / N) - m * m
    h = jnp.maximum((y[0] - m) * lax.rsqrt(v + 1e-5) * g[...] + beta[...], 0.0)
    h = h * _rowmask(j)
    u3 = lax.dot_general(Wl3[...], Wout[...], (((0,), (1,)), ((), ())),
                         precision=HIGH)
    v3 = lax.dot_general(Wr3[...], Wout[...], (((0,), (1,)), ((), ())),
                         precision=HIGH)
    z_out[0] = lax.dot_general(h, u3, (((1,), (0,)), ((), ())), precision=HIGH)
    w_out[0] = lax.dot_general(h, v3, (((1,), (0,)), ((), ())), precision=HIGH)


def _wspec():
    return pl.BlockSpec((H, H), lambda t, j: (0, 0))


def _vspec():
    return pl.BlockSpec((H,), lambda t, j: (0,))


def _blkspec(w):
    return pl.BlockSpec((1, RB, w), lambda t, j: (t, j, 0))


def _statspec():
    return pl.BlockSpec((1, 2, H), lambda t, j: (t, 0, 0))


def _lin_call(h_spec):
    return pl.pallas_call(
        _lin_body,
        grid=(2, NRB),
        in_specs=[_blkspec(H), _blkspec(1), h_spec,
                  _wspec(), _vspec(), _wspec(), _vspec(), _vspec()],
        out_specs=[_blkspec(H), _statspec()],
        out_shape=[jax.ShapeDtypeStruct((2, NP, H), F32),
                   jax.ShapeDtypeStruct((2, 2, H), F32)],
    )


def _final_body(t, deg, w, batch, bl3, Wout, bout, out):
    c3 = jnp.sum(bl3[...] * Wout[0, :])
    node = 0.5 * (t[0] / jnp.maximum(deg[0], 1.0)
                  + t[1] / jnp.maximum(deg[1], 1.0)
                  + w[0] + w[1]) + c3
    b = batch[...]
    gi = lax.broadcasted_iota(jnp.int32, (G, 1), 0)
    acc = jnp.zeros((G, 1), F32)
    for g in range(G):
        mask = b == g
        sg = jnp.sum(jnp.where(mask, node, 0.0))
        cg = jnp.sum(jnp.where(mask, 1.0, 0.0))
        acc = acc + jnp.where(gi == g, sg / jnp.maximum(cg, 1.0), 0.0)
    out[...] = acc + bout[...]


def kernel(x, edge_index_sc, edge_index_fc, batch,
           Wl1, bl1, Wr1, g1, beta1,
           Wl2, bl2, Wr2, g2, beta2,
           Wl3, bl3, Wr3, Wout, bout):
    edges = jnp.stack([edge_index_sc, edge_index_fc]).reshape(4 * E)
    x2 = jnp.pad(x, ((0, NP - N), (0, 0)))[None]

    sums1, deg = _agg1(x2.reshape(NP, H), edges)
    deg3 = deg.reshape(2, NP, 1)

    y1, st1 = _lin_call(pl.BlockSpec((1, RB, H), lambda t, j: (0, j, 0)))(
        sums1, deg3, x2, Wl1, bl1, Wr1, g1, beta1)
    bn1 = pl.pallas_call(
        _bn_body,
        grid=(2, NRB),
        in_specs=[_blkspec(H), _statspec(), _vspec(), _vspec()],
        out_specs=_blkspec(H),
        out_shape=jax.ShapeDtypeStruct((2, NP, H), F32),
    )
    h1 = bn1(y1, st1, g1, beta1)

    sums2 = _agg2(h1.reshape(2 * NP, H), edges)

    y2, st2 = _lin_call(_blkspec(H))(
        sums2, deg3, h1, Wl2, bl2, Wr2, g2, beta2)
    bnproj = pl.pallas_call(
        _bnproj_body,
        grid=(2, NRB),
        in_specs=[_blkspec(H), _statspec(), _vspec(), _vspec(),
                  _wspec(), _wspec(),
                  pl.BlockSpec((1, H), lambda t, j: (0, 0))],
        out_specs=[_blkspec(1), _blkspec(1)],
        out_shape=[jax.ShapeDtypeStruct((2, NP, 1), F32),
                   jax.ShapeDtypeStruct((2, NP, 1), F32)],
    )
    z, w = bnproj(y2, st2, g2, beta2, Wl3, Wr3, Wout)

    t = _sagg(z.reshape(2 * NP), edges)

    batch_p = jnp.pad(batch, (0, NP - N), constant_values=G)
    final = pl.pallas_call(
        _final_body,
        out_shape=jax.ShapeDtypeStruct((G, 1), F32),
    )
    return final(t.reshape(2, NP // H, H), deg.reshape(2, NP // H, H),
                 w.reshape(2, NP // H, H), batch_p.reshape(NP // H, H),
                 bl3, Wout, bout.reshape(1, 1))
